# baseline (speedup 1.0000x reference)
import functools

import jax
import jax.numpy as jnp
from jax import lax
from jax.experimental import pallas as pl
from jax.experimental.pallas import tpu as pltpu
from jax.experimental.pallas import tpu_sc as plsc

VOCAB1 = 1001
D = 16
NCAT = 26
B = 16384

_NC = 2
_NS = 16
_NW = _NC * _NS
_BPW = B // _NW

def _sc_gather_body(item_id_hbm, cat_hbm, table_hbm, w1cat_hbm,
                    emb_out, catrow_out,
                    idx_v, rows_v, idx2_v, rows2_v, sem1, sem2):
    wid = lax.axis_index("s") * _NC + lax.axis_index("c")
    base = wid * _BPW
    pltpu.sync_copy(item_id_hbm.at[pl.ds(base, _BPW)], idx_v)
    cp1 = pltpu.async_copy(table_hbm.at[idx_v], rows_v, sem1)
    pltpu.sync_copy(cat_hbm.at[pl.ds(base, _BPW)], idx2_v)
    cp2 = pltpu.async_copy(w1cat_hbm.at[idx2_v], rows2_v, sem2)
    cp1.wait()
    pltpu.sync_copy(rows_v, emb_out.at[pl.ds(base, _BPW)])
    cp2.wait()
    pltpu.sync_copy(rows2_v, catrow_out.at[pl.ds(base, _BPW)])


@functools.lru_cache(maxsize=1)
def _get_sc_gather():
    mesh = plsc.VectorSubcoreMesh(core_axis_name="c", subcore_axis_name="s")
    return pl.kernel(
        _sc_gather_body,
        out_type=[jax.ShapeDtypeStruct((B, D), jnp.float32),
                  jax.ShapeDtypeStruct((B, D), jnp.float32)],
        mesh=mesh,
        scratch_types=[pltpu.VMEM((_BPW,), jnp.int32),
                       pltpu.VMEM((_BPW, D), jnp.float32),
                       pltpu.VMEM((_BPW,), jnp.int32),
                       pltpu.VMEM((_BPW, D), jnp.float32),
                       pltpu.SemaphoreType.DMA,
                       pltpu.SemaphoreType.DMA],
        compiler_params=pltpu.CompilerParams(use_tc_tiling_on_sc=False),
    )


_BLK = 2048
_INV = 1.0 / (1.0 + 1e-6)


def _mlp_body(emb_ref, cat_ref, s_ref, w1a_ref, w1s_ref, b1_ref,
              w2_ref, b2_ref, out_ref):
    acc = jnp.dot(emb_ref[:], w1a_ref[:], preferred_element_type=jnp.float32)
    acc = acc + cat_ref[:]
    s = s_ref[:] * jnp.float32(_INV)
    acc = acc + jnp.dot(s, w1s_ref[:], preferred_element_type=jnp.float32)
    acc = acc + b1_ref[:]
    h = jnp.maximum(acc, jnp.float32(0.0))
    out_ref[:] = (jnp.dot(h, w2_ref[:], preferred_element_type=jnp.float32)
                  + b2_ref[:])


_mlp = pl.pallas_call(
    _mlp_body,
    grid=(B // _BLK,),
    in_specs=[
        pl.BlockSpec((_BLK, D), lambda i: (i, 0)),
        pl.BlockSpec((_BLK, D), lambda i: (i, 0)),
        pl.BlockSpec((_BLK, 3), lambda i: (i, 0)),
        pl.BlockSpec((D, D), lambda i: (0, 0)),
        pl.BlockSpec((3, D), lambda i: (0, 0)),
        pl.BlockSpec((1, D), lambda i: (0, 0)),
        pl.BlockSpec((D, D), lambda i: (0, 0)),
        pl.BlockSpec((1, D), lambda i: (0, 0)),
    ],
    out_specs=pl.BlockSpec((_BLK, D), lambda i: (i, 0)),
    out_shape=jax.ShapeDtypeStruct((B, D), jnp.float32),
)


def kernel(item_id, category, view_count, click_count, title_length,
           item_table, W1, b1, W2, b2):
    item_id = item_id.astype(jnp.int32)
    category = category.astype(jnp.int32)
    w1a = W1[:D]
    w1cat = W1[D:D + NCAT]
    w1s = W1[D + NCAT:]
    emb, catrow = _get_sc_gather()(item_id, category, item_table, w1cat)
    s = jnp.stack([view_count, click_count, title_length], axis=1)
    return _mlp(emb, catrow, s, w1a, w1s, b1.reshape(1, D), W2,
                b2.reshape(1, D))


# speedup vs baseline: 1.6823x; 1.6823x over previous
import functools

import jax
import jax.numpy as jnp
from jax import lax
from jax.experimental import pallas as pl
from jax.experimental.pallas import tpu as pltpu
from jax.experimental.pallas import tpu_sc as plsc

VOCAB1 = 1001
D = 16
NCAT = 26
B = 16384

_NC = 2
_NS = 16
_NW = _NC * _NS
_BPW = B // _NW


def _sc_gather_body(item_id_hbm, table_hbm, emb_out, idx_v, rows_v, sem):
    wid = lax.axis_index("s") * _NC + lax.axis_index("c")
    base = wid * _BPW
    pltpu.sync_copy(item_id_hbm.at[pl.ds(base, _BPW)], idx_v)
    cp = pltpu.async_copy(table_hbm.at[idx_v], rows_v, sem)
    cp.wait()
    pltpu.sync_copy(rows_v, emb_out.at[pl.ds(base, _BPW)])


@functools.lru_cache(maxsize=1)
def _get_sc_gather():
    mesh = plsc.VectorSubcoreMesh(core_axis_name="c", subcore_axis_name="s")
    return pl.kernel(
        _sc_gather_body,
        out_type=jax.ShapeDtypeStruct((B, D), jnp.float32),
        mesh=mesh,
        scratch_types=[pltpu.VMEM((_BPW,), jnp.int32),
                       pltpu.VMEM((_BPW, D), jnp.float32),
                       pltpu.SemaphoreType.DMA],
        compiler_params=pltpu.CompilerParams(use_tc_tiling_on_sc=False),
    )


_BLK = 2048
_INV = 1.0 / (1.0 + 1e-6)


def _mlp_body(emb_ref, cat_ref, vn_ref, cn_ref, tn_ref, w1_ref,
              b1_ref, w2_ref, b2_ref, out_ref):
    acc = jnp.dot(emb_ref[:], w1_ref[:D, :],
                  preferred_element_type=jnp.float32)
    cat = cat_ref[:].reshape(_BLK, 1)
    oh = (cat == lax.broadcasted_iota(jnp.int32, (_BLK, NCAT), 1))
    acc = acc + jnp.dot(oh.astype(jnp.float32), w1_ref[D:D + NCAT, :],
                        preferred_element_type=jnp.float32)
    inv = jnp.float32(_INV)
    acc = acc + (vn_ref[:].reshape(_BLK, 1) * inv) * w1_ref[D + NCAT, :]
    acc = acc + (cn_ref[:].reshape(_BLK, 1) * inv) * w1_ref[D + NCAT + 1, :]
    acc = acc + (tn_ref[:].reshape(_BLK, 1) * inv) * w1_ref[D + NCAT + 2, :]
    acc = acc + b1_ref[:]
    h = jnp.maximum(acc, jnp.float32(0.0))
    out_ref[:] = (jnp.dot(h, w2_ref[:], preferred_element_type=jnp.float32)
                  + b2_ref[:])


_mlp_grid_spec = dict(
    grid=(B // _BLK,),
    in_specs=[
        pl.BlockSpec((_BLK, D), lambda i: (i, 0)),
        pl.BlockSpec((_BLK,), lambda i: (i,)),
        pl.BlockSpec((_BLK,), lambda i: (i,)),
        pl.BlockSpec((_BLK,), lambda i: (i,)),
        pl.BlockSpec((_BLK,), lambda i: (i,)),
        pl.BlockSpec((D + NCAT + 3, D), lambda i: (0, 0)),
        pl.BlockSpec((1, D), lambda i: (0, 0)),
        pl.BlockSpec((D, D), lambda i: (0, 0)),
        pl.BlockSpec((1, D), lambda i: (0, 0)),
    ],
    out_specs=pl.BlockSpec((_BLK, D), lambda i: (i, 0)),
    out_shape=jax.ShapeDtypeStruct((B, D), jnp.float32),
)

_mlp = pl.pallas_call(_mlp_body, **_mlp_grid_spec)


def kernel(item_id, category, view_count, click_count, title_length,
           item_table, W1, b1, W2, b2):
    emb = _get_sc_gather()(item_id.astype(jnp.int32), item_table)
    return _mlp(emb, category.astype(jnp.int32), view_count, click_count,
                title_length, W1, b1.reshape(1, D), W2, b2.reshape(1, D))


# speedup vs baseline: 1.7281x; 1.0272x over previous
import functools

import jax
import jax.numpy as jnp
from jax import lax
from jax.experimental import pallas as pl
from jax.experimental.pallas import tpu as pltpu
from jax.experimental.pallas import tpu_sc as plsc

VOCAB1 = 1001
D = 16
NCAT = 26
B = 16384

_NC = 2
_NS = 16
_NW = _NC * _NS
_BPW = B // _NW


def _sc_gather_body(item_id_hbm, table_hbm, emb_out, idx_v, rows_v, sem):
    wid = lax.axis_index("s") * _NC + lax.axis_index("c")
    base = wid * _BPW
    pltpu.sync_copy(item_id_hbm.at[pl.ds(base, _BPW)], idx_v)
    cp = pltpu.async_copy(table_hbm.at[idx_v], rows_v, sem)
    cp.wait()
    pltpu.sync_copy(rows_v, emb_out.at[pl.ds(base, _BPW)])


@functools.lru_cache(maxsize=1)
def _get_sc_gather():
    mesh = plsc.VectorSubcoreMesh(core_axis_name="c", subcore_axis_name="s")
    return pl.kernel(
        _sc_gather_body,
        out_type=jax.ShapeDtypeStruct((B, D), jnp.float32),
        mesh=mesh,
        scratch_types=[pltpu.VMEM((_BPW,), jnp.int32),
                       pltpu.VMEM((_BPW, D), jnp.float32),
                       pltpu.SemaphoreType.DMA],
        compiler_params=pltpu.CompilerParams(use_tc_tiling_on_sc=False),
    )


_BLK = 4096
_INV = 1.0 / (1.0 + 1e-6)


def _mlp_body(emb_ref, cat_ref, vn_ref, cn_ref, tn_ref, w1_ref,
              b1_ref, w2_ref, b2_ref, out_ref):
    acc = jnp.dot(emb_ref[:], w1_ref[:D, :],
                  preferred_element_type=jnp.float32)
    cat = cat_ref[:].reshape(_BLK, 1)
    oh = (cat == lax.broadcasted_iota(jnp.int32, (_BLK, NCAT), 1))
    acc = acc + jnp.dot(oh.astype(jnp.float32), w1_ref[D:D + NCAT, :],
                        preferred_element_type=jnp.float32)
    inv = jnp.float32(_INV)
    acc = acc + (vn_ref[:].reshape(_BLK, 1) * inv) * w1_ref[D + NCAT, :]
    acc = acc + (cn_ref[:].reshape(_BLK, 1) * inv) * w1_ref[D + NCAT + 1, :]
    acc = acc + (tn_ref[:].reshape(_BLK, 1) * inv) * w1_ref[D + NCAT + 2, :]
    acc = acc + b1_ref[:][None, :]
    h = jnp.maximum(acc, jnp.float32(0.0))
    out_ref[:] = (jnp.dot(h, w2_ref[:], preferred_element_type=jnp.float32)
                  + b2_ref[:][None, :])


_mlp_grid_spec = dict(
    grid=(B // _BLK,),
    in_specs=[
        pl.BlockSpec((_BLK, D), lambda i: (i, 0)),
        pl.BlockSpec((_BLK,), lambda i: (i,)),
        pl.BlockSpec((_BLK,), lambda i: (i,)),
        pl.BlockSpec((_BLK,), lambda i: (i,)),
        pl.BlockSpec((_BLK,), lambda i: (i,)),
        pl.BlockSpec((D + NCAT + 3, D), lambda i: (0, 0)),
        pl.BlockSpec((D,), lambda i: (0,)),
        pl.BlockSpec((D, D), lambda i: (0, 0)),
        pl.BlockSpec((D,), lambda i: (0,)),
    ],
    out_specs=pl.BlockSpec((_BLK, D), lambda i: (i, 0)),
    out_shape=jax.ShapeDtypeStruct((B, D), jnp.float32),
)

_mlp = pl.pallas_call(_mlp_body, **_mlp_grid_spec)


def kernel(item_id, category, view_count, click_count, title_length,
           item_table, W1, b1, W2, b2):
    emb = _get_sc_gather()(item_id.astype(jnp.int32), item_table)
    return _mlp(emb, category.astype(jnp.int32), view_count, click_count,
                title_length, W1, b1, W2, b2)
